# baseline (speedup 1.0000x reference)
import math

import jax
import jax.numpy as jnp
from jax import lax
from jax.experimental import pallas as pl
from jax.experimental.pallas import tpu as pltpu
from jax.experimental.pallas import tpu_sc as plsc

_N_NODES = 50000
_N_VARS = 128
_BATCH = 1024
_HALF_LOG_2PI = 0.5 * math.log(2.0 * math.pi)
_BIG = 1e30

_NC, _NS = 2, 16
_OWN = 3328
_N_PAD = _NS * _OWN
_CHUNK = 2000
_N_CHUNKS = _N_NODES // _CHUNK
_STEPS = _CHUNK // 16
_GSTEPS = _OWN // 16

_W = 4096
_N_BLOCKS = (_N_NODES + _W - 1) // _W


def _sc_permute_body(idx_hbm, par_hbm, sco_hbm,
                     mu_out, sg_out, sc_out,
                     idx_v, inv_loc, buf, res, sem, vsem):
    c = lax.axis_index("c")
    s = lax.axis_index("s")
    lo = s * _OWN
    iota = lax.iota(jnp.int32, 16)

    @pl.when(c == 0)
    def _stage_params():
        pltpu.async_copy(par_hbm, buf, vsem)

    @pl.when(c == 1)
    def _stage_scopes():
        pltpu.async_copy(sco_hbm, buf.at[pl.ds(0, _N_NODES)], vsem)

    def stage(k, b):
        return pltpu.async_copy(
            idx_hbm.at[pl.ds(k * _CHUNK, _CHUNK)], idx_v.at[b], sem)

    pending = stage(0, 0)
    for k in range(_N_CHUNKS):
        pending.wait()
        b = k % 2
        if k + 1 < _N_CHUNKS:
            pending = stage(k + 1, (k + 1) % 2)
        base = k * _CHUNK

        @plsc.parallel_loop(0, _STEPS, 1, unroll=8)
        def _chunk_scan(i):
            sl = pl.ds(i * 16, 16)
            pos = idx_v.at[b][sl] - lo
            m = pos.astype(jnp.uint32) < jnp.uint32(_OWN)
            plsc.store_scatter(inv_loc, [pos], base + i * 16 + iota, mask=m)

    @pl.when(c == 0)
    def _gather_params():
        pltpu.make_async_copy(par_hbm, buf, vsem).wait()

        @plsc.parallel_loop(0, _GSTEPS, 1, unroll=8)
        def _gather_mu(i):
            sl = pl.ds(i * 16, 16)
            valid = lo + i * 16 + iota < _N_NODES
            inv16 = jnp.where(valid, inv_loc[sl], 0)
            res[sl] = plsc.load_gather(buf, [inv16 * 2])

        pltpu.sync_copy(res, mu_out.at[pl.ds(lo, _OWN)])

        @plsc.parallel_loop(0, _GSTEPS, 1, unroll=8)
        def _gather_sg(i):
            sl = pl.ds(i * 16, 16)
            valid = lo + i * 16 + iota < _N_NODES
            inv16 = jnp.where(valid, inv_loc[sl], 0)
            res[sl] = plsc.load_gather(buf, [inv16 * 2 + 1])

        pltpu.sync_copy(res, sg_out.at[pl.ds(lo, _OWN)])

    @pl.when(c == 1)
    def _gather_scopes():
        pltpu.make_async_copy(
            sco_hbm, buf.at[pl.ds(0, _N_NODES)], vsem).wait()

        @plsc.parallel_loop(0, _GSTEPS, 1, unroll=8)
        def _gather_sc(i):
            sl = pl.ds(i * 16, 16)
            valid = lo + i * 16 + iota < _N_NODES
            inv16 = jnp.where(valid, inv_loc[sl], 0)
            res[sl] = plsc.load_gather(buf, [inv16])

        pltpu.sync_copy(res, sc_out.at[pl.ds(lo, _OWN)])


def _sc_permute(idx, par_flat, sco_f32):
    f = pl.kernel(
        _sc_permute_body,
        out_type=(jax.ShapeDtypeStruct((_N_PAD,), jnp.float32),) * 3,
        mesh=plsc.VectorSubcoreMesh(
            core_axis_name="c", subcore_axis_name="s"),
        compiler_params=pltpu.CompilerParams(
            use_tc_tiling_on_sc=False, needs_layout_passes=False),
        scratch_types=[
            pltpu.VMEM((2, _CHUNK), jnp.int32),
            pltpu.VMEM((_OWN,), jnp.int32),
            pltpu.VMEM((2 * _N_NODES,), jnp.float32),
            pltpu.VMEM((_OWN,), jnp.float32),
            pltpu.SemaphoreType.DMA,
            pltpu.SemaphoreType.DMA,
        ],
    )
    return f(idx, par_flat, sco_f32)


def _tc_body(x_ref, mu_ref, sg_ref, sc_ref, out_ref):
    xb = x_ref[...]
    xb = jnp.where(jnp.isnan(xb), jnp.float32(_BIG), xb)
    mu = mu_ref[...]
    sigma = jnp.maximum(sg_ref[...], jnp.float32(1e-5))
    scope = lax.bitcast_convert_type(sc_ref[...], jnp.int32)
    iot = lax.broadcasted_iota(jnp.int32, (_N_VARS, _W), 0)
    oneh = jnp.where(iot == scope, jnp.float32(1.0), jnp.float32(0.0))
    val = jnp.dot(xb, oneh, preferred_element_type=jnp.float32)
    z = (val - mu) / sigma
    lld = -0.5 * z * z - jnp.log(sigma) - _HALF_LOG_2PI
    out_ref[...] = jnp.where(val >= jnp.float32(_BIG * 0.5),
                             jnp.float32(0.0), lld)


def _tc_logprob(x, mu_p, sg_p, sc_p):
    row = pl.BlockSpec((1, _W), lambda i: (0, i))
    return pl.pallas_call(
        _tc_body,
        grid=(_N_BLOCKS,),
        in_specs=[
            pl.BlockSpec((_BATCH, _N_VARS), lambda i: (0, 0)),
            row, row, row,
        ],
        out_specs=pl.BlockSpec((_BATCH, _W), lambda i: (0, i)),
        out_shape=jax.ShapeDtypeStruct((_BATCH, _N_NODES), jnp.float32),
        compiler_params=pltpu.CompilerParams(
            dimension_semantics=("arbitrary",)),
    )(x, mu_p.reshape(1, _N_PAD), sg_p.reshape(1, _N_PAD),
      sc_p.reshape(1, _N_PAD))


def kernel(x, params, scopes, idx):
    par_flat = params.reshape(-1)
    sco_f32 = lax.bitcast_convert_type(scopes, jnp.float32)
    mu_p, sg_p, sc_p = _sc_permute(idx, par_flat, sco_f32)
    return _tc_logprob(x, mu_p, sg_p, sc_p)


# speedup vs baseline: 1.0372x; 1.0372x over previous
import math

import jax
import jax.numpy as jnp
from jax import lax
from jax.experimental import pallas as pl
from jax.experimental.pallas import tpu as pltpu
from jax.experimental.pallas import tpu_sc as plsc

_N_NODES = 50000
_N_VARS = 128
_BATCH = 1024
_HALF_LOG_2PI = 0.5 * math.log(2.0 * math.pi)
_BIG = 1e30

_NC, _NS = 2, 16
_OWN = 3328
_N_PAD = _NS * _OWN
_CHUNK = 10000
_N_CHUNKS = _N_NODES // _CHUNK
_STEPS = _CHUNK // 16
_GSTEPS = _OWN // 16

_W = 4096
_N_BLOCKS = (_N_NODES + _W - 1) // _W


def _sc_permute_body(idx_hbm, par_hbm, sco_hbm,
                     mu_out, sg_out, sc_out,
                     idx_v, inv_loc, buf, res, sem, vsem):
    c = lax.axis_index("c")
    s = lax.axis_index("s")
    lo = s * _OWN
    iota = lax.iota(jnp.int32, 16)

    @pl.when(c == 0)
    def _stage_params():
        pltpu.async_copy(par_hbm, buf, vsem)

    @pl.when(c == 1)
    def _stage_scopes():
        pltpu.async_copy(sco_hbm, buf.at[pl.ds(0, _N_NODES)], vsem)

    def stage(k, b):
        return pltpu.async_copy(
            idx_hbm.at[pl.ds(k * _CHUNK, _CHUNK)], idx_v.at[b], sem)

    pending = stage(0, 0)
    for k in range(_N_CHUNKS):
        pending.wait()
        b = k % 2
        if k + 1 < _N_CHUNKS:
            pending = stage(k + 1, (k + 1) % 2)
        base = k * _CHUNK

        @plsc.parallel_loop(0, _STEPS, 1, unroll=8)
        def _chunk_scan(i):
            sl = pl.ds(i * 16, 16)
            pos = idx_v.at[b][sl] - lo
            m = pos.astype(jnp.uint32) < jnp.uint32(_OWN)
            plsc.store_scatter(inv_loc, [pos], base + i * 16 + iota, mask=m)

    @pl.when(c == 0)
    def _gather_params():
        pltpu.make_async_copy(par_hbm, buf, vsem).wait()

        @plsc.parallel_loop(0, _GSTEPS, 1, unroll=8)
        def _gather_mu(i):
            sl = pl.ds(i * 16, 16)
            valid = lo + i * 16 + iota < _N_NODES
            inv16 = jnp.where(valid, inv_loc[sl], 0)
            res[sl] = plsc.load_gather(buf, [inv16 * 2])

        pltpu.sync_copy(res, mu_out.at[pl.ds(lo, _OWN)])

        @plsc.parallel_loop(0, _GSTEPS, 1, unroll=8)
        def _gather_sg(i):
            sl = pl.ds(i * 16, 16)
            valid = lo + i * 16 + iota < _N_NODES
            inv16 = jnp.where(valid, inv_loc[sl], 0)
            res[sl] = plsc.load_gather(buf, [inv16 * 2 + 1])

        pltpu.sync_copy(res, sg_out.at[pl.ds(lo, _OWN)])

    @pl.when(c == 1)
    def _gather_scopes():
        pltpu.make_async_copy(
            sco_hbm, buf.at[pl.ds(0, _N_NODES)], vsem).wait()

        @plsc.parallel_loop(0, _GSTEPS, 1, unroll=8)
        def _gather_sc(i):
            sl = pl.ds(i * 16, 16)
            valid = lo + i * 16 + iota < _N_NODES
            inv16 = jnp.where(valid, inv_loc[sl], 0)
            res[sl] = plsc.load_gather(buf, [inv16])

        pltpu.sync_copy(res, sc_out.at[pl.ds(lo, _OWN)])


def _sc_permute(idx, par_flat, sco_f32):
    f = pl.kernel(
        _sc_permute_body,
        out_type=(jax.ShapeDtypeStruct((_N_PAD,), jnp.float32),) * 3,
        mesh=plsc.VectorSubcoreMesh(
            core_axis_name="c", subcore_axis_name="s"),
        compiler_params=pltpu.CompilerParams(
            use_tc_tiling_on_sc=False, needs_layout_passes=False),
        scratch_types=[
            pltpu.VMEM((2, _CHUNK), jnp.int32),
            pltpu.VMEM((_OWN,), jnp.int32),
            pltpu.VMEM((2 * _N_NODES,), jnp.float32),
            pltpu.VMEM((_OWN,), jnp.float32),
            pltpu.SemaphoreType.DMA,
            pltpu.SemaphoreType.DMA,
        ],
    )
    return f(idx, par_flat, sco_f32)


def _tc_body(x_ref, mu_ref, sg_ref, sc_ref, out_ref):
    xb = x_ref[...]
    xb = jnp.where(jnp.isnan(xb), jnp.float32(_BIG), xb)
    mu = mu_ref[...]
    sigma = jnp.maximum(sg_ref[...], jnp.float32(1e-5))
    scope = lax.bitcast_convert_type(sc_ref[...], jnp.int32)
    iot = lax.broadcasted_iota(jnp.int32, (_N_VARS, _W), 0)
    oneh = jnp.where(iot == scope, jnp.float32(1.0), jnp.float32(0.0))
    val = jnp.dot(xb, oneh, preferred_element_type=jnp.float32)
    z = (val - mu) / sigma
    lld = -0.5 * z * z - jnp.log(sigma) - _HALF_LOG_2PI
    out_ref[...] = jnp.where(val >= jnp.float32(_BIG * 0.5),
                             jnp.float32(0.0), lld)


def _tc_logprob(x, mu_p, sg_p, sc_p):
    row = pl.BlockSpec((1, _W), lambda i: (0, i))
    return pl.pallas_call(
        _tc_body,
        grid=(_N_BLOCKS,),
        in_specs=[
            pl.BlockSpec((_BATCH, _N_VARS), lambda i: (0, 0)),
            row, row, row,
        ],
        out_specs=pl.BlockSpec((_BATCH, _W), lambda i: (0, i)),
        out_shape=jax.ShapeDtypeStruct((_BATCH, _N_NODES), jnp.float32),
        compiler_params=pltpu.CompilerParams(
            dimension_semantics=("arbitrary",)),
    )(x, mu_p.reshape(1, _N_PAD), sg_p.reshape(1, _N_PAD),
      sc_p.reshape(1, _N_PAD))


def kernel(x, params, scopes, idx):
    par_flat = params.reshape(-1)
    sco_f32 = lax.bitcast_convert_type(scopes, jnp.float32)
    mu_p, sg_p, sc_p = _sc_permute(idx, par_flat, sco_f32)
    return _tc_logprob(x, mu_p, sg_p, sc_p)


# speedup vs baseline: 1.0892x; 1.0501x over previous
import math

import jax
import jax.numpy as jnp
from jax import lax
from jax.experimental import pallas as pl
from jax.experimental.pallas import tpu as pltpu
from jax.experimental.pallas import tpu_sc as plsc

_N_NODES = 50000
_N_VARS = 128
_BATCH = 1024
_HALF_LOG_2PI = 0.5 * math.log(2.0 * math.pi)
_BIG = 1e30

_NC, _NS = 2, 16
_NW = _NC * _NS
_OWN = 1664
_N_PAD = _NW * _OWN
_CHUNK = 4096
_N_CHUNKS = _N_PAD // _CHUNK
_STEPS = _CHUNK // 16
_GSTEPS = _OWN // 16

_W = 4096
_N_BLOCKS = (_N_NODES + _W - 1) // _W


def _sc_permute_body(idx_hbm, mu_hbm, sg_hbm, sc_hbm,
                     mu_out, sg_out, sc_out,
                     idx_v, inv_loc, val_full, res, sem, vsem):
    c = lax.axis_index("c")
    s = lax.axis_index("s")
    lo = (s * _NC + c) * _OWN
    hi = lo + _OWN

    val_cps = [pltpu.async_copy(mu_hbm, val_full.at[0], vsem),
               pltpu.async_copy(sg_hbm, val_full.at[1], vsem)]

    def stage(k, buf):
        return pltpu.async_copy(
            idx_hbm.at[pl.ds(k * _CHUNK, _CHUNK)], idx_v.at[buf], sem)

    pending = stage(0, 0)
    for k in range(_N_CHUNKS):
        pending.wait()
        buf = k % 2
        if k + 1 < _N_CHUNKS:
            pending = stage(k + 1, (k + 1) % 2)
        base = k * _CHUNK

        @plsc.parallel_loop(0, _STEPS, 1, unroll=8)
        def _chunk_scan(i):
            sl = pl.ds(i * 16, 16)
            pos = idx_v.at[buf][sl] - lo
            m = pos.astype(jnp.uint32) < jnp.uint32(_OWN)
            nids = base + i * 16 + lax.iota(jnp.int32, 16)
            plsc.store_scatter(inv_loc, [pos], nids, mask=m)

    for a, out in enumerate((mu_out, sg_out, sc_out)):
        val_cps[a].wait()
        vbuf = a % 2

        @plsc.parallel_loop(0, _GSTEPS, 1, unroll=8)
        def _gather(i):
            sl = pl.ds(i * 16, 16)
            res[sl] = plsc.load_gather(val_full.at[vbuf], [inv_loc[sl]])

        if a == 0:
            val_cps.append(pltpu.async_copy(sc_hbm, val_full.at[0], vsem))
        pltpu.sync_copy(res, out.at[pl.ds(lo, _OWN)])


def _sc_permute(idx_pad, mu_pad, sg_pad, sc_pad):
    f = pl.kernel(
        _sc_permute_body,
        out_type=(jax.ShapeDtypeStruct((_N_PAD,), jnp.float32),) * 3,
        mesh=plsc.VectorSubcoreMesh(
            core_axis_name="c", subcore_axis_name="s"),
        compiler_params=pltpu.CompilerParams(
            use_tc_tiling_on_sc=False, needs_layout_passes=False),
        scratch_types=[
            pltpu.VMEM((2, _CHUNK), jnp.int32),
            pltpu.VMEM((_OWN,), jnp.int32),
            pltpu.VMEM((2, _N_PAD), jnp.float32),
            pltpu.VMEM((_OWN,), jnp.float32),
            pltpu.SemaphoreType.DMA,
            pltpu.SemaphoreType.DMA,
        ],
    )
    return f(idx_pad, mu_pad, sg_pad, sc_pad)


def _tc_body(x_ref, mu_ref, sg_ref, sc_ref, out_ref):
    xb = x_ref[...]
    xb = jnp.where(jnp.isnan(xb), jnp.float32(_BIG), xb)
    mu = mu_ref[...]
    sigma = jnp.maximum(sg_ref[...], jnp.float32(1e-5))
    scope = sc_ref[...].astype(jnp.int32)
    iot = lax.broadcasted_iota(jnp.int32, (_N_VARS, _W), 0)
    oneh = jnp.where(iot == scope, jnp.float32(1.0), jnp.float32(0.0))
    val = jnp.dot(xb, oneh, preferred_element_type=jnp.float32)
    z = (val - mu) / sigma
    lld = -0.5 * z * z - jnp.log(sigma) - _HALF_LOG_2PI
    out_ref[...] = jnp.where(val >= jnp.float32(_BIG * 0.5),
                             jnp.float32(0.0), lld)


def _tc_logprob(x, mu_p, sg_p, sc_p):
    row = pl.BlockSpec((1, _W), lambda i: (0, i))
    return pl.pallas_call(
        _tc_body,
        grid=(_N_BLOCKS,),
        in_specs=[
            pl.BlockSpec((_BATCH, _N_VARS), lambda i: (0, 0)),
            row, row, row,
        ],
        out_specs=pl.BlockSpec((_BATCH, _W), lambda i: (0, i)),
        out_shape=jax.ShapeDtypeStruct((_BATCH, _N_NODES), jnp.float32),
        compiler_params=pltpu.CompilerParams(
            dimension_semantics=("arbitrary",)),
    )(x, mu_p.reshape(1, _N_PAD), sg_p.reshape(1, _N_PAD),
      sc_p.reshape(1, _N_PAD))


def kernel(x, params, scopes, idx):
    pad = _N_PAD - _N_NODES
    idx_pad = jnp.concatenate(
        [idx, jnp.arange(_N_NODES, _N_PAD, dtype=jnp.int32)])
    mu_pad = jnp.pad(params[:, 0], (0, pad))
    sg_pad = jnp.pad(params[:, 1], (0, pad))
    sc_pad = jnp.pad(scopes.astype(jnp.float32), (0, pad))
    mu_p, sg_p, sc_p = _sc_permute(idx_pad, mu_pad, sg_pad, sc_pad)
    return _tc_logprob(x, mu_p, sg_p, sc_p)
